# Initial kernel scaffold; baseline (speedup 1.0000x reference)
import jax
import jax.numpy as jnp
from jax.experimental import pallas as pl


def kernel(y_true, y_pred, weights):
    raise NotImplementedError("write your pallas kernel here")


# speedup vs baseline: 14.3163x; 14.3163x over previous
import functools

import jax
import jax.numpy as jnp
from jax import lax
from jax.experimental import pallas as pl
from jax.experimental.pallas import tpu as pltpu
from jax.experimental.pallas import tpu_sc as plsc

N_SAMPLES = 32
N_PIX = 512 * 512
TOPK = int(round(0.3 * N_PIX))

LANES = 16
NBINS = 1024
JBINS = NBINS // LANES * LANES
SHIFT = 18
BASE = 3424
CHUNK = 8192
HIST_WORDS = LANES * NBINS


def _sc_partial_sums(n_pix, chunk, k, interpret=False):
    nchunk = n_pix // chunk
    vpc = chunk // LANES
    jrows = NBINS // LANES
    kf = float(k)
    mesh = plsc.VectorSubcoreMesh(
        core_axis_name="c", subcore_axis_name="s", num_cores=2, num_subcores=16)

    @functools.partial(
        pl.kernel,
        out_type=jax.ShapeDtypeStruct((N_SAMPLES, LANES), jnp.float32),
        mesh=mesh,
        interpret=interpret,
        compiler_params=pltpu.CompilerParams(needs_layout_passes=False),
        scratch_types=[
            pltpu.VMEM((2, chunk), jnp.float32),
            pltpu.VMEM((2, chunk), jnp.float32),
            pltpu.VMEM((2, chunk), jnp.float32),
            pltpu.VMEM((HIST_WORDS,), jnp.float32),
            pltpu.VMEM((HIST_WORDS,), jnp.float32),
            pltpu.VMEM((LANES,), jnp.float32),
            pltpu.SemaphoreType.DMA,
            pltpu.SemaphoreType.DMA,
        ],
    )
    def body(yt_hbm, yp_hbm, w_hbm, out_hbm, byt, byp, bw,
             hist_c, hist_s, res_v, sem0, sem1):
        wid = lax.axis_index("s") * 2 + lax.axis_index("c")
        sems = (sem0, sem1)
        zero16 = jnp.zeros((LANES,), jnp.float32)
        ones16 = jnp.ones((LANES,), jnp.float32)
        dl_iota = lax.iota(jnp.int32, LANES)
        dl_off = dl_iota * NBINS

        def zbody(v, carry):
            hist_c[pl.ds(v * LANES, LANES)] = zero16
            hist_s[pl.ds(v * LANES, LANES)] = zero16
            return carry
        lax.fori_loop(0, HIST_WORDS // LANES, zbody, 0)

        def chunk_copies(c, slot):
            src = pl.ds(c * chunk, chunk)
            return [
                pltpu.make_async_copy(yt_hbm.at[wid, src], byt.at[slot], sems[slot]),
                pltpu.make_async_copy(yp_hbm.at[wid, src], byp.at[slot], sems[slot]),
                pltpu.make_async_copy(w_hbm.at[wid, src], bw.at[slot], sems[slot]),
            ]

        for cp in chunk_copies(0, 0):
            cp.start()

        for c in range(nchunk):
            slot = c & 1
            if c + 1 < nchunk:
                for cp in chunk_copies(c + 1, (c + 1) & 1):
                    cp.start()
            for cp in chunk_copies(c, slot):
                cp.wait()

            def vbody(i, carry):
                off = i * LANES
                yt = byt[slot, pl.ds(off, LANES)]
                yp = byp[slot, pl.ds(off, LANES)]
                wv = bw[slot, pl.ds(off, LANES)]
                loss = lax.abs(yt - yp) * wv
                bits = lax.bitcast_convert_type(loss, jnp.int32)
                b = lax.shift_right_arithmetic(bits, SHIFT) - BASE
                b = lax.min(lax.max(b, 0), NBINS - 1)
                j = lax.shift_left(b & (jrows - 1), 4)
                l = lax.shift_right_logical(b, 6)
                idx = dl_off + j + l
                plsc.addupdate_scatter(hist_c, [idx], ones16)
                plsc.addupdate_scatter(hist_s, [idx], loss)
                return carry
            lax.fori_loop(0, vpc, vbody, 0, unroll=4)

        def rbody(j, carry):
            t_c, t_s = carry
            acc_c = hist_c[pl.ds(j * LANES, LANES)]
            acc_s = hist_s[pl.ds(j * LANES, LANES)]
            for dl in range(1, LANES):
                o = dl * NBINS + j * LANES
                acc_c = acc_c + hist_c[pl.ds(o, LANES)]
                acc_s = acc_s + hist_s[pl.ds(o, LANES)]
            hist_c[pl.ds(j * LANES, LANES)] = acc_c
            hist_s[pl.ds(j * LANES, LANES)] = acc_s
            return (t_c + acc_c, t_s + acc_s)
        t_c, t_s = lax.fori_loop(0, jrows, rbody, (zero16, zero16))

        def suffix_excl(t):
            r = lax.rev(t, (0,))
            cs = plsc.cumsum(r)
            return lax.rev(cs - r, (0,))
        above_c = suffix_excl(t_c)
        above_s = suffix_excl(t_s)

        def sbody(i, carry):
            run_c, run_s, res = carry
            j = jrows - 1 - i
            cvec = hist_c[pl.ds(j * LANES, LANES)]
            svec = hist_s[pl.ds(j * LANES, LANES)]
            g_next = above_c + run_c
            s_next = above_s + run_s
            g = g_next + cvec
            cross = jnp.logical_and(g >= kf, g_next < kf)
            mean = svec / cvec
            contrib = jnp.where(cross, s_next + (kf - g_next) * mean, 0.0)
            return (run_c + cvec, run_s + svec, res + contrib)
        _, _, res = lax.fori_loop(0, jrows, sbody, (zero16, zero16, zero16))

        res_v[...] = res
        pltpu.sync_copy(res_v, out_hbm.at[wid])

    return body


def _tc_finish(x_ref, o_ref):
    o_ref[...] = jnp.full((1, 1), jnp.sum(x_ref[...]) * (1.0 / (N_SAMPLES * TOPK)),
                          jnp.float32)


@jax.jit
def kernel(y_true, y_pred, weights):
    yt = y_true.reshape(N_SAMPLES, N_PIX)
    yp = y_pred.reshape(N_SAMPLES, N_PIX)
    w = weights.reshape(N_SAMPLES, N_PIX)
    partial = _sc_partial_sums(N_PIX, CHUNK, TOPK)(yt, yp, w)
    out = pl.pallas_call(
        _tc_finish,
        out_shape=jax.ShapeDtypeStruct((1, 1), jnp.float32),
    )(partial)
    return out.reshape(())


# speedup vs baseline: 15.5336x; 1.0850x over previous
import functools

import jax
import jax.numpy as jnp
from jax import lax
from jax.experimental import pallas as pl
from jax.experimental.pallas import tpu as pltpu
from jax.experimental.pallas import tpu_sc as plsc

N_SAMPLES = 32
N_PIX = 512 * 512
TOPK = int(round(0.3 * N_PIX))

LANES = 16
NBINS = 1024
JBINS = NBINS // LANES * LANES
SHIFT = 18
BASE = 3424
CHUNK = 8192
HIST_WORDS = LANES * NBINS


def _sc_partial_sums(n_pix, chunk, k, interpret=False):
    nchunk = n_pix // chunk
    vpc = chunk // LANES
    jrows = NBINS // LANES
    kf = float(k)
    mesh = plsc.VectorSubcoreMesh(
        core_axis_name="c", subcore_axis_name="s", num_cores=2, num_subcores=16)

    @functools.partial(
        pl.kernel,
        out_type=jax.ShapeDtypeStruct((N_SAMPLES, LANES), jnp.float32),
        mesh=mesh,
        interpret=interpret,
        compiler_params=pltpu.CompilerParams(needs_layout_passes=False),
        scratch_types=[
            pltpu.VMEM((2, chunk), jnp.float32),
            pltpu.VMEM((2, chunk), jnp.float32),
            pltpu.VMEM((2, chunk), jnp.float32),
            pltpu.VMEM((HIST_WORDS,), jnp.float32),
            pltpu.VMEM((HIST_WORDS,), jnp.float32),
            pltpu.VMEM((LANES,), jnp.float32),
            pltpu.SemaphoreType.DMA,
            pltpu.SemaphoreType.DMA,
        ],
    )
    def body(yt_hbm, yp_hbm, w_hbm, out_hbm, byt, byp, bw,
             hist_c, hist_s, res_v, sem0, sem1):
        wid = lax.axis_index("s") * 2 + lax.axis_index("c")
        sems = (sem0, sem1)
        zero16 = jnp.zeros((LANES,), jnp.float32)
        ones16 = jnp.ones((LANES,), jnp.float32)
        dl_iota = lax.iota(jnp.int32, LANES)
        dl_off = dl_iota * NBINS

        def zbody(v, carry):
            hist_c[pl.ds(v * LANES, LANES)] = zero16
            hist_s[pl.ds(v * LANES, LANES)] = zero16
            return carry
        lax.fori_loop(0, HIST_WORDS // LANES, zbody, 0)

        def chunk_copies(c, slot):
            src = pl.ds(c * chunk, chunk)
            return [
                pltpu.make_async_copy(yt_hbm.at[wid, src], byt.at[slot], sems[slot]),
                pltpu.make_async_copy(yp_hbm.at[wid, src], byp.at[slot], sems[slot]),
                pltpu.make_async_copy(w_hbm.at[wid, src], bw.at[slot], sems[slot]),
            ]

        for cp in chunk_copies(0, 0):
            cp.start()

        for c in range(nchunk):
            slot = c & 1
            if c + 1 < nchunk:
                for cp in chunk_copies(c + 1, (c + 1) & 1):
                    cp.start()
            for cp in chunk_copies(c, slot):
                cp.wait()

            def vbody(i, carry):
                off = i * LANES
                yt = byt[slot, pl.ds(off, LANES)]
                yp = byp[slot, pl.ds(off, LANES)]
                wv = bw[slot, pl.ds(off, LANES)]
                loss = lax.abs(yt - yp) * wv
                bits = lax.bitcast_convert_type(loss, jnp.int32)
                b = lax.shift_right_arithmetic(bits, SHIFT) - BASE
                b = lax.min(lax.max(b, 0), NBINS - 1)
                j = lax.shift_left(b & (jrows - 1), 4)
                l = lax.shift_right_logical(b, 6)
                rot = (l + dl_iota) & (LANES - 1)
                idx = dl_off + j + rot
                plsc.addupdate_scatter(hist_c, [idx], ones16)
                plsc.addupdate_scatter(hist_s, [idx], loss)
                return carry
            lax.fori_loop(0, vpc, vbody, 0, unroll=4)

        def rbody(j, carry):
            t_c, t_s = carry
            acc_c = hist_c[pl.ds(j * LANES, LANES)]
            acc_s = hist_s[pl.ds(j * LANES, LANES)]
            for dl in range(1, LANES):
                o = dl * NBINS + j * LANES
                perm = (dl_iota + dl) & (LANES - 1)
                acc_c = acc_c + hist_c[pl.ds(o, LANES)][perm]
                acc_s = acc_s + hist_s[pl.ds(o, LANES)][perm]
            hist_c[pl.ds(j * LANES, LANES)] = acc_c
            hist_s[pl.ds(j * LANES, LANES)] = acc_s
            return (t_c + acc_c, t_s + acc_s)
        t_c, t_s = lax.fori_loop(0, jrows, rbody, (zero16, zero16))

        def suffix_excl(t):
            r = lax.rev(t, (0,))
            cs = plsc.cumsum(r)
            return lax.rev(cs - r, (0,))
        above_c = suffix_excl(t_c)
        above_s = suffix_excl(t_s)

        def sbody(i, carry):
            run_c, run_s, res = carry
            j = jrows - 1 - i
            cvec = hist_c[pl.ds(j * LANES, LANES)]
            svec = hist_s[pl.ds(j * LANES, LANES)]
            g_next = above_c + run_c
            s_next = above_s + run_s
            g = g_next + cvec
            cross = jnp.logical_and(g >= kf, g_next < kf)
            mean = svec / cvec
            contrib = jnp.where(cross, s_next + (kf - g_next) * mean, 0.0)
            return (run_c + cvec, run_s + svec, res + contrib)
        _, _, res = lax.fori_loop(0, jrows, sbody, (zero16, zero16, zero16))

        res_v[...] = res
        pltpu.sync_copy(res_v, out_hbm.at[wid])

    return body


def _tc_finish(x_ref, o_ref):
    o_ref[...] = jnp.full((1, 1), jnp.sum(x_ref[...]) * (1.0 / (N_SAMPLES * TOPK)),
                          jnp.float32)


@jax.jit
def kernel(y_true, y_pred, weights):
    yt = y_true.reshape(N_SAMPLES, N_PIX)
    yp = y_pred.reshape(N_SAMPLES, N_PIX)
    w = weights.reshape(N_SAMPLES, N_PIX)
    partial = _sc_partial_sums(N_PIX, CHUNK, TOPK)(yt, yp, w)
    out = pl.pallas_call(
        _tc_finish,
        out_shape=jax.ShapeDtypeStruct((1, 1), jnp.float32),
    )(partial)
    return out.reshape(())


# speedup vs baseline: 31.7938x; 2.0468x over previous
import functools

import jax
import jax.numpy as jnp
from jax import lax
from jax.experimental import pallas as pl
from jax.experimental.pallas import tpu as pltpu
from jax.experimental.pallas import tpu_sc as plsc

N_SAMPLES = 32
N_PIX = 512 * 512
TOPK = int(round(0.3 * N_PIX))

LANES = 16
NBINS = 1024
JBINS = NBINS // LANES * LANES
SHIFT = 18
BASE = 3424
CHUNK = 8192
HIST_WORDS = LANES * NBINS


def _sc_partial_sums(n_pix, chunk, k, interpret=False):
    nchunk = n_pix // chunk
    vpc = chunk // LANES
    jrows = NBINS // LANES
    kf = float(k)
    mesh = plsc.VectorSubcoreMesh(
        core_axis_name="c", subcore_axis_name="s", num_cores=2, num_subcores=16)

    @functools.partial(
        pl.kernel,
        out_type=jax.ShapeDtypeStruct((N_SAMPLES, LANES), jnp.float32),
        mesh=mesh,
        interpret=interpret,
        compiler_params=pltpu.CompilerParams(needs_layout_passes=False),
        scratch_types=[
            pltpu.VMEM((2, chunk), jnp.float32),
            pltpu.VMEM((2, chunk), jnp.float32),
            pltpu.VMEM((2, chunk), jnp.float32),
            pltpu.VMEM((HIST_WORDS,), jnp.float32),
            pltpu.VMEM((HIST_WORDS,), jnp.float32),
            pltpu.VMEM((LANES,), jnp.float32),
            pltpu.SemaphoreType.DMA,
            pltpu.SemaphoreType.DMA,
        ],
    )
    def body(yt_hbm, yp_hbm, w_hbm, out_hbm, byt, byp, bw,
             hist_c, hist_s, res_v, sem0, sem1):
        wid = lax.axis_index("s") * 2 + lax.axis_index("c")
        sems = (sem0, sem1)
        zero16 = jnp.zeros((LANES,), jnp.float32)
        ones16 = jnp.ones((LANES,), jnp.float32)
        dl_iota = lax.iota(jnp.int32, LANES)
        dl_off = dl_iota * NBINS

        def zbody(v, carry):
            hist_c[pl.ds(v * LANES, LANES)] = zero16
            hist_s[pl.ds(v * LANES, LANES)] = zero16
            return carry
        lax.fori_loop(0, HIST_WORDS // LANES, zbody, 0)

        def chunk_copies(c, slot):
            src = pl.ds(c * chunk, chunk)
            return [
                pltpu.make_async_copy(yt_hbm.at[wid, src], byt.at[slot], sems[slot]),
                pltpu.make_async_copy(yp_hbm.at[wid, src], byp.at[slot], sems[slot]),
                pltpu.make_async_copy(w_hbm.at[wid, src], bw.at[slot], sems[slot]),
            ]

        for cp in chunk_copies(0, 0):
            cp.start()

        for c in range(nchunk):
            slot = c & 1
            if c + 1 < nchunk:
                for cp in chunk_copies(c + 1, (c + 1) & 1):
                    cp.start()
            for cp in chunk_copies(c, slot):
                cp.wait()

            @plsc.parallel_loop(0, vpc, unroll=4)
            def vbody(i):
                off = i * LANES
                yt = byt[slot, pl.ds(off, LANES)]
                yp = byp[slot, pl.ds(off, LANES)]
                wv = bw[slot, pl.ds(off, LANES)]
                loss = lax.abs(yt - yp) * wv
                bits = lax.bitcast_convert_type(loss, jnp.int32)
                b = lax.shift_right_arithmetic(bits, SHIFT) - BASE
                b = lax.min(lax.max(b, 0), NBINS - 1)
                j = lax.shift_left(b & (jrows - 1), 4)
                l = lax.shift_right_logical(b, 6)
                rot = (l + dl_iota) & (LANES - 1)
                idx = dl_off + j + rot
                plsc.addupdate_scatter(hist_c, [idx], ones16)
                plsc.addupdate_scatter(hist_s, [idx], loss)

        def rbody(j, carry):
            t_c, t_s = carry
            acc_c = hist_c[pl.ds(j * LANES, LANES)]
            acc_s = hist_s[pl.ds(j * LANES, LANES)]
            for dl in range(1, LANES):
                o = dl * NBINS + j * LANES
                perm = (dl_iota + dl) & (LANES - 1)
                acc_c = acc_c + hist_c[pl.ds(o, LANES)][perm]
                acc_s = acc_s + hist_s[pl.ds(o, LANES)][perm]
            hist_c[pl.ds(j * LANES, LANES)] = acc_c
            hist_s[pl.ds(j * LANES, LANES)] = acc_s
            return (t_c + acc_c, t_s + acc_s)
        t_c, t_s = lax.fori_loop(0, jrows, rbody, (zero16, zero16))

        def suffix_excl(t):
            r = lax.rev(t, (0,))
            cs = plsc.cumsum(r)
            return lax.rev(cs - r, (0,))
        above_c = suffix_excl(t_c)
        above_s = suffix_excl(t_s)

        def sbody(i, carry):
            run_c, run_s, res = carry
            j = jrows - 1 - i
            cvec = hist_c[pl.ds(j * LANES, LANES)]
            svec = hist_s[pl.ds(j * LANES, LANES)]
            g_next = above_c + run_c
            s_next = above_s + run_s
            g = g_next + cvec
            cross = jnp.logical_and(g >= kf, g_next < kf)
            mean = svec / cvec
            contrib = jnp.where(cross, s_next + (kf - g_next) * mean, 0.0)
            return (run_c + cvec, run_s + svec, res + contrib)
        _, _, res = lax.fori_loop(0, jrows, sbody, (zero16, zero16, zero16))

        res_v[...] = res
        pltpu.sync_copy(res_v, out_hbm.at[wid])

    return body


def _tc_finish(x_ref, o_ref):
    o_ref[...] = jnp.full((1, 1), jnp.sum(x_ref[...]) * (1.0 / (N_SAMPLES * TOPK)),
                          jnp.float32)


@jax.jit
def kernel(y_true, y_pred, weights):
    yt = y_true.reshape(N_SAMPLES, N_PIX)
    yp = y_pred.reshape(N_SAMPLES, N_PIX)
    w = weights.reshape(N_SAMPLES, N_PIX)
    partial = _sc_partial_sums(N_PIX, CHUNK, TOPK)(yt, yp, w)
    out = pl.pallas_call(
        _tc_finish,
        out_shape=jax.ShapeDtypeStruct((1, 1), jnp.float32),
    )(partial)
    return out.reshape(())
